# Initial kernel scaffold; baseline (speedup 1.0000x reference)
import jax
import jax.numpy as jnp
from jax.experimental import pallas as pl


def kernel(unknown, known, unknow_feats, known_feats, W1, b1, g1, beta1, W2, b2, g2, beta2):
    raise NotImplementedError("write your pallas kernel here")


# speedup vs baseline: 7.6455x; 7.6455x over previous
import functools

import jax
import jax.numpy as jnp
from jax import lax
from jax.experimental import pallas as pl
from jax.experimental.pallas import tpu as pltpu
from jax.experimental.pallas import tpu_sc as plsc

B, N, M, C1, C2 = 8, 4096, 1024, 256, 256
BN = B * N
TN = 512
TM = 1024

NC, NS = 2, 16
NW = NC * NS
Q = BN // NW
CH = 16
NCHUNK = Q // CH


def _knn_body(u_ref, k_ref, idx_ref, w_ref):
    b = pl.program_id(0)
    u = u_ref[0]
    k = k_ref[0]
    cross = lax.dot_general(u, k, (((1,), (1,)), ((), ())),
                            preferred_element_type=jnp.float32,
                            precision=lax.Precision.HIGHEST)
    un = jnp.sum(u * u, axis=1, keepdims=True)
    kn = jnp.sum(k * k, axis=1, keepdims=True).reshape(1, M)
    d2 = jnp.maximum(un + kn - 2.0 * cross, 0.0)

    iota = lax.broadcasted_iota(jnp.int32, (TN, M), 1)
    dists, idxs = [], []
    for _ in range(3):
        m = jnp.min(d2, axis=1, keepdims=True)
        i = jnp.min(jnp.where(d2 == m, iota, jnp.int32(M)),
                    axis=1, keepdims=True)
        d2 = jnp.where(iota == i, jnp.float32(jnp.inf), d2)
        dists.append(m)
        idxs.append(i)

    r = [1.0 / (d + 1e-10) for d in dists]
    norm = r[0] + r[1] + r[2]
    w_ref[0] = jnp.concatenate([x / norm for x in r], axis=1)
    idx_ref[0] = jnp.concatenate(idxs, axis=1) + b * M


def _knn(unknown, known):
    return pl.pallas_call(
        _knn_body,
        grid=(B, N // TN),
        in_specs=[
            pl.BlockSpec((1, TN, 3), lambda b, i: (b, i, 0)),
            pl.BlockSpec((1, M, 3), lambda b, i: (b, 0, 0)),
        ],
        out_specs=[
            pl.BlockSpec((1, TN, 3), lambda b, i: (b, i, 0)),
            pl.BlockSpec((1, TN, 3), lambda b, i: (b, i, 0)),
        ],
        out_shape=[
            jax.ShapeDtypeStruct((B, N, 3), jnp.int32),
            jax.ShapeDtypeStruct((B, N, 3), jnp.float32),
        ],
    )(unknown, known)


def _sc_interp_body(kf_hbm, idx_hbm, w_hbm, out_hbm, idx_v, w_v, rows_v,
                    out_v, sem):
    wid = lax.axis_index("s") * NC + lax.axis_index("c")

    def chunk(t, _):
        base_r = wid * Q + t * CH
        base_i = base_r * 3
        pltpu.sync_copy(idx_hbm.at[pl.ds(base_i, CH * 3)], idx_v)
        pltpu.sync_copy(w_hbm.at[pl.ds(base_i, CH * 3)], w_v)
        pltpu.async_copy(kf_hbm.at[idx_v], rows_v, sem).wait()
        for rr in range(CH):
            for g in range(C2 // 16):
                s = pl.ds(g * 16, 16)
                acc = w_v[3 * rr, :] * rows_v[3 * rr, s]
                acc = acc + w_v[3 * rr + 1, :] * rows_v[3 * rr + 1, s]
                acc = acc + w_v[3 * rr + 2, :] * rows_v[3 * rr + 2, s]
                out_v[rr, s] = acc
        pltpu.sync_copy(out_v, out_hbm.at[pl.ds(base_r, CH)])
        return _

    lax.fori_loop(0, NCHUNK, chunk, None)


def _sc_interp(kf_flat, idx_flat, wexp):
    run = pl.kernel(
        _sc_interp_body,
        mesh=plsc.VectorSubcoreMesh(core_axis_name="c", subcore_axis_name="s"),
        out_type=jax.ShapeDtypeStruct((BN, C2), jnp.float32),
        scratch_types=[
            pltpu.VMEM((CH * 3,), jnp.int32),
            pltpu.VMEM((CH * 3, 16), jnp.float32),
            pltpu.VMEM((CH * 3, C2), jnp.float32),
            pltpu.VMEM((CH, C2), jnp.float32),
            pltpu.SemaphoreType.DMA,
        ],
    )
    return run(kf_flat, idx_flat, wexp)


def _mm_stats_body(x1_ref, x2_ref, wa_ref, wb_ref, b_ref, sc_ref, sh_ref,
                   h_ref, s_ref, q_ref, *, relu_in):
    x1 = x1_ref[...]
    if relu_in:
        x1 = jnp.maximum(x1 * sc_ref[...] + sh_ref[...], 0.0)
    h = jnp.dot(x1, wa_ref[...], preferred_element_type=jnp.float32,
                precision=lax.Precision.HIGHEST)
    if x2_ref is not None:
        h = h + jnp.dot(x2_ref[...], wb_ref[...],
                        preferred_element_type=jnp.float32,
                        precision=lax.Precision.HIGHEST)
    h = h + b_ref[...]
    h_ref[...] = h

    @pl.when(pl.program_id(0) == 0)
    def _():
        s_ref[...] = jnp.zeros_like(s_ref)
        q_ref[...] = jnp.zeros_like(q_ref)

    s_ref[...] += jnp.sum(h, axis=0, keepdims=True)
    q_ref[...] += jnp.sum(h * h, axis=0, keepdims=True)


def _pass_a(interp, unk, w1at, w1bt, b1):
    body = functools.partial(_mm_stats_body, relu_in=False)

    def wrapped(x1, x2, wa, wb, bb, h, s, q):
        body(x1, x2, wa, wb, bb, None, None, h, s, q)

    row = pl.BlockSpec((TM, C2), lambda i: (i, 0))
    full = pl.BlockSpec((C2, C2), lambda i: (0, 0))
    vec = pl.BlockSpec((1, C2), lambda i: (0, 0))
    return pl.pallas_call(
        wrapped,
        grid=(BN // TM,),
        in_specs=[row, row, full, full, vec],
        out_specs=[row, vec, vec],
        out_shape=[
            jax.ShapeDtypeStruct((BN, C2), jnp.float32),
            jax.ShapeDtypeStruct((1, C2), jnp.float32),
            jax.ShapeDtypeStruct((1, C2), jnp.float32),
        ],
    )(interp, unk, w1at, w1bt, b1)


def _pass_b(h1, w2t, b2, scale1, shift1):
    def wrapped(x1, wa, bb, sc, sh, h, s, q):
        _mm_stats_body(x1, None, wa, None, bb, sc, sh, h, s, q, relu_in=True)

    row = pl.BlockSpec((TM, C2), lambda i: (i, 0))
    full = pl.BlockSpec((C2, C2), lambda i: (0, 0))
    vec = pl.BlockSpec((1, C2), lambda i: (0, 0))
    return pl.pallas_call(
        wrapped,
        grid=(BN // TM,),
        in_specs=[row, full, vec, vec, vec],
        out_specs=[row, vec, vec],
        out_shape=[
            jax.ShapeDtypeStruct((BN, C2), jnp.float32),
            jax.ShapeDtypeStruct((1, C2), jnp.float32),
            jax.ShapeDtypeStruct((1, C2), jnp.float32),
        ],
    )(h1, w2t, b2, scale1, shift1)


def _pass_c_body(h_ref, sc_ref, sh_ref, o_ref):
    o_ref[...] = jnp.maximum(h_ref[...] * sc_ref[...] + sh_ref[...], 0.0)


def _pass_c(h2, scale2, shift2):
    row = pl.BlockSpec((TM, C2), lambda i: (i, 0))
    vec = pl.BlockSpec((1, C2), lambda i: (0, 0))
    return pl.pallas_call(
        _pass_c_body,
        grid=(BN // TM,),
        in_specs=[row, vec, vec],
        out_specs=row,
        out_shape=jax.ShapeDtypeStruct((BN, C2), jnp.float32),
    )(h2, scale2, shift2)


def _affine(s, q, g, beta, eps=1e-5):
    mu = s / BN
    var = q / BN - mu * mu
    scale = g.reshape(1, -1) * lax.rsqrt(var + eps)
    shift = beta.reshape(1, -1) - mu * scale
    return scale, shift


def kernel(unknown, known, unknow_feats, known_feats, W1, b1, g1, beta1,
           W2, b2, g2, beta2):
    idx, w = _knn(unknown, known)

    idx_flat = idx.reshape(BN * 3)
    wexp = jnp.broadcast_to(w.reshape(BN * 3, 1), (BN * 3, 16))
    kf_flat = known_feats.reshape(B * M, C2)
    interp = _sc_interp(kf_flat, idx_flat, wexp)

    unk = unknow_feats.reshape(BN, C1)
    w1at = W1[:, :C2].T
    w1bt = W1[:, C2:].T
    h1, s1, q1 = _pass_a(interp, unk, w1at, w1bt, b1.reshape(1, C2))
    scale1, shift1 = _affine(s1, q1, g1, beta1)
    h2, s2, q2 = _pass_b(h1, W2.T, b2.reshape(1, C2), scale1, shift1)
    scale2, shift2 = _affine(s2, q2, g2, beta2)
    out = _pass_c(h2, scale2, shift2)
    return out.reshape(B, N, C2)
